# baseline (speedup 1.0000x reference)
import functools

import jax
import jax.numpy as jnp
from jax import lax
from jax.experimental import pallas as pl
from jax.experimental.pallas import tpu as pltpu
from jax.experimental.pallas import tpu_sc as plsc

_SIZE = 400
_NCAM = 6
_C = 256
_M = 39200
_NPTS = _NCAM * _M
_NCELL = _SIZE * _SIZE

_NC, _NS = 2, 16
_NW = _NC * _NS
_CHUNK = 128
_NCHUNK = 58
_PER_W = _NCHUNK * _CHUNK
_NPAD = _NW * _PER_W
_ACC_PAD = _NCELL + 8


_CK = 64


def _s1_body(w_ref, td_ref, o_ref):
    k = pl.program_id(1)
    part = lax.dot_general(w_ref[...], td_ref[0],
                           (((0,), (0,)), ((), ())),
                           preferred_element_type=jnp.float32)

    @pl.when(k == 0)
    def _():
        o_ref[0] = part

    @pl.when(k != 0)
    def _():
        o_ref[0] += part


def _stage1(td2, Wt):
    return pl.pallas_call(
        _s1_body,
        grid=(_NCAM, _C // _CK),
        in_specs=[
            pl.BlockSpec((_CK, 1), lambda n, k: (k, 0)),
            pl.BlockSpec((1, _CK, _M), lambda n, k: (n, k, 0)),
        ],
        out_specs=pl.BlockSpec((1, 1, _M), lambda n, k: (n, 0, 0)),
        out_shape=jax.ShapeDtypeStruct((_NCAM, 1, _M), jnp.float32),
    )(Wt, td2)


@functools.cache
def _make_sc_kernel():
    mesh = plsc.VectorSubcoreMesh(core_axis_name="c", subcore_axis_name="s")

    @functools.partial(
        pl.kernel,
        mesh=mesh,
        out_type=[
            jax.ShapeDtypeStruct((_NC, _NCELL), jnp.float32),
            jax.ShapeDtypeStruct((_NC, _NCELL), jnp.float32),
        ],
        scratch_types=[
            pltpu.VMEM((_NCHUNK, _CHUNK), jnp.int32),
            pltpu.VMEM((_NCHUNK, _CHUNK), jnp.int32),
            pltpu.VMEM((_NCHUNK, _CHUNK), jnp.float32),
            pltpu.VMEM((_NCHUNK, _CHUNK), jnp.float32),
            pltpu.VMEM_SHARED((_ACC_PAD,), jnp.float32),
            pltpu.VMEM_SHARED((_ACC_PAD,), jnp.float32),
            pltpu.SemaphoreType.DMA,
            pltpu.SemaphoreType.DMA,
        ],
    )
    def sc_kernel(s_hbm, src_hbm, dst_hbm, zeros_hbm, ones_hbm,
                  acc_out, cnt_out,
                  idx_s, idx_d, vals, ones_v, acc_sh, cnt_sh, sem_g, sem_s):
        cid = lax.axis_index("c")
        sid = lax.axis_index("s")
        wid = sid * _NC + cid

        pltpu.sync_copy(src_hbm.at[wid], idx_s)
        pltpu.sync_copy(dst_hbm.at[wid], idx_d)
        pltpu.sync_copy(ones_hbm, ones_v)

        @pl.when(sid == 0)
        def _():
            pltpu.sync_copy(zeros_hbm, acc_sh)
            pltpu.sync_copy(zeros_hbm, cnt_sh)

        def _gather(j, c):
            pltpu.async_copy(s_hbm.at[idx_s.at[j]], vals.at[j], sem_g)
            return c

        lax.fori_loop(0, _NCHUNK, _gather, 0)
        pltpu.make_async_copy(ones_hbm, vals, sem_g).wait()

        plsc.subcore_barrier()

        def _scatter(j, c):
            pltpu.async_copy(vals.at[j], acc_sh.at[idx_d.at[j]], sem_s, add=True)
            pltpu.async_copy(ones_v.at[j], cnt_sh.at[idx_d.at[j]], sem_s, add=True)
            return c

        lax.fori_loop(0, _NCHUNK, _scatter, 0)
        pltpu.make_async_copy(ones_hbm, vals, sem_s).wait()
        pltpu.make_async_copy(ones_hbm, ones_v, sem_s).wait()

        plsc.subcore_barrier()

        @pl.when(sid == 0)
        def _():
            pltpu.sync_copy(acc_sh.at[pl.ds(0, _NCELL)], acc_out.at[cid])
            pltpu.sync_copy(cnt_sh.at[pl.ds(0, _NCELL)], cnt_out.at[cid])

    return sc_kernel


def _s3_body(a_ref, c_ref, b_ref, o_ref):
    a = a_ref[0] + a_ref[1]
    t = c_ref[0] + c_ref[1]
    denom = jnp.where(t >= 1.0, t, 1.0)
    o_ref[...] = a / denom + b_ref[0]


def _stage3(acc, cnt, b):
    return pl.pallas_call(
        _s3_body,
        in_specs=[
            pl.BlockSpec((_NC, _SIZE, _SIZE), lambda: (0, 0, 0)),
            pl.BlockSpec((_NC, _SIZE, _SIZE), lambda: (0, 0, 0)),
            pl.BlockSpec(memory_space=pltpu.SMEM),
        ],
        out_specs=pl.BlockSpec((_SIZE, _SIZE), lambda: (0, 0)),
        out_shape=jax.ShapeDtypeStruct((_SIZE, _SIZE), jnp.float32),
    )(acc, cnt, b)


def kernel(td_feats, coords, ids, W, b):
    td2 = td_feats.reshape(_NCAM, _C, _M)
    s = _stage1(td2, W.reshape(_C, 1))
    s_flat = s.reshape(_NPTS)

    cam_off = (jnp.arange(_NCAM, dtype=jnp.int32) * _M)[:, None]
    src_idx = (cam_off + ids[:, 1, :] * 200 + ids[:, 0, :]).reshape(_NPTS)
    dst_idx = (coords[:, 0, :] * _SIZE + coords[:, 1, :]).reshape(_NPTS)
    npad = _NPAD - _NPTS
    src_idx = jnp.concatenate(
        [src_idx, jnp.zeros((npad,), jnp.int32)]).reshape(_NW, _NCHUNK, _CHUNK)
    dst_idx = jnp.concatenate(
        [dst_idx, jnp.full((npad,), _NCELL, jnp.int32)]).reshape(_NW, _NCHUNK, _CHUNK)

    zeros = jnp.zeros((_ACC_PAD,), jnp.float32)
    ones = jnp.ones((_NCHUNK, _CHUNK), jnp.float32)
    acc, cnt = _make_sc_kernel()(s_flat, src_idx, dst_idx, zeros, ones)

    logits = _stage3(acc.reshape(_NC, _SIZE, _SIZE),
                     cnt.reshape(_NC, _SIZE, _SIZE), b)
    return logits[None, None, :, :]


# speedup vs baseline: 1.0216x; 1.0216x over previous
import functools

import jax
import jax.numpy as jnp
from jax import lax
from jax.experimental import pallas as pl
from jax.experimental.pallas import tpu as pltpu
from jax.experimental.pallas import tpu_sc as plsc

_SIZE = 400
_NCAM = 6
_C = 256
_M = 39200
_NPTS = _NCAM * _M
_NCELL = _SIZE * _SIZE

_NC, _NS = 2, 16
_NW = _NC * _NS
_CHUNK = 128
_NCHUNK = 58
_PER_W = _NCHUNK * _CHUNK
_NPAD = _NW * _PER_W
_ACC_PAD = _NCELL + 8


_CK = 64


def _s1_body(w_ref, td_ref, o_ref):
    k = pl.program_id(1)
    part = lax.dot_general(w_ref[...], td_ref[0],
                           (((0,), (0,)), ((), ())),
                           preferred_element_type=jnp.float32)

    @pl.when(k == 0)
    def _():
        o_ref[0] = part

    @pl.when(k != 0)
    def _():
        o_ref[0] += part


def _stage1(td2, Wt):
    return pl.pallas_call(
        _s1_body,
        grid=(_NCAM, _C // _CK),
        in_specs=[
            pl.BlockSpec((_CK, 1), lambda n, k: (k, 0)),
            pl.BlockSpec((1, _CK, _M), lambda n, k: (n, k, 0)),
        ],
        out_specs=pl.BlockSpec((1, 1, _M), lambda n, k: (n, 0, 0)),
        out_shape=jax.ShapeDtypeStruct((_NCAM, 1, _M), jnp.float32),
    )(Wt, td2)


@functools.cache
def _make_sc_kernel():
    mesh = plsc.VectorSubcoreMesh(core_axis_name="c", subcore_axis_name="s")

    @functools.partial(
        pl.kernel,
        mesh=mesh,
        out_type=[
            jax.ShapeDtypeStruct((_NC, _NCELL), jnp.float32),
            jax.ShapeDtypeStruct((_NC, _NCELL), jnp.float32),
        ],
        scratch_types=[
            pltpu.VMEM((_NCHUNK, _CHUNK), jnp.int32),
            pltpu.VMEM((_NCHUNK, _CHUNK), jnp.int32),
            pltpu.VMEM((_NCHUNK, _CHUNK), jnp.float32),
            pltpu.VMEM((_NCHUNK, _CHUNK), jnp.float32),
            pltpu.VMEM_SHARED((_ACC_PAD,), jnp.float32),
            pltpu.VMEM_SHARED((_ACC_PAD,), jnp.float32),
            pltpu.VMEM_SHARED((_NPTS,), jnp.float32),
            pltpu.SemaphoreType.DMA,
            pltpu.SemaphoreType.DMA,
        ],
    )
    def sc_kernel(s_hbm, src_hbm, dst_hbm, zeros_hbm, ones_hbm,
                  acc_out, cnt_out,
                  idx_s, idx_d, vals, ones_v, acc_sh, cnt_sh, s_sh,
                  sem_g, sem_s):
        cid = lax.axis_index("c")
        sid = lax.axis_index("s")
        wid = sid * _NC + cid

        pltpu.sync_copy(src_hbm.at[wid], idx_s)
        pltpu.sync_copy(dst_hbm.at[wid], idx_d)
        pltpu.sync_copy(ones_hbm, ones_v)

        @pl.when(sid == 0)
        def _():
            pltpu.sync_copy(zeros_hbm, acc_sh)
            pltpu.sync_copy(zeros_hbm, cnt_sh)
            pltpu.sync_copy(s_hbm, s_sh)

        plsc.subcore_barrier()

        def _gather(j, c):
            pltpu.async_copy(s_sh.at[idx_s.at[j]], vals.at[j], sem_g)
            return c

        lax.fori_loop(0, _NCHUNK, _gather, 0)
        pltpu.make_async_copy(ones_hbm, vals, sem_g).wait()

        def _scatter(j, c):
            pltpu.async_copy(vals.at[j], acc_sh.at[idx_d.at[j]], sem_s, add=True)
            pltpu.async_copy(ones_v.at[j], cnt_sh.at[idx_d.at[j]], sem_s, add=True)
            return c

        lax.fori_loop(0, _NCHUNK, _scatter, 0)
        pltpu.make_async_copy(ones_hbm, vals, sem_s).wait()
        pltpu.make_async_copy(ones_hbm, ones_v, sem_s).wait()

        plsc.subcore_barrier()

        @pl.when(sid == 0)
        def _():
            pltpu.sync_copy(acc_sh.at[pl.ds(0, _NCELL)], acc_out.at[cid])
            pltpu.sync_copy(cnt_sh.at[pl.ds(0, _NCELL)], cnt_out.at[cid])

    return sc_kernel


def _s3_body(a_ref, c_ref, b_ref, o_ref):
    a = a_ref[0] + a_ref[1]
    t = c_ref[0] + c_ref[1]
    denom = jnp.where(t >= 1.0, t, 1.0)
    o_ref[...] = a / denom + b_ref[0]


def _stage3(acc, cnt, b):
    return pl.pallas_call(
        _s3_body,
        in_specs=[
            pl.BlockSpec((_NC, _SIZE, _SIZE), lambda: (0, 0, 0)),
            pl.BlockSpec((_NC, _SIZE, _SIZE), lambda: (0, 0, 0)),
            pl.BlockSpec(memory_space=pltpu.SMEM),
        ],
        out_specs=pl.BlockSpec((_SIZE, _SIZE), lambda: (0, 0)),
        out_shape=jax.ShapeDtypeStruct((_SIZE, _SIZE), jnp.float32),
    )(acc, cnt, b)


def kernel(td_feats, coords, ids, W, b):
    td2 = td_feats.reshape(_NCAM, _C, _M)
    s = _stage1(td2, W.reshape(_C, 1))
    s_flat = s.reshape(_NPTS)

    cam_off = (jnp.arange(_NCAM, dtype=jnp.int32) * _M)[:, None]
    src_idx = (cam_off + ids[:, 1, :] * 200 + ids[:, 0, :]).reshape(_NPTS)
    dst_idx = (coords[:, 0, :] * _SIZE + coords[:, 1, :]).reshape(_NPTS)
    npad = _NPAD - _NPTS
    src_idx = jnp.concatenate(
        [src_idx, jnp.zeros((npad,), jnp.int32)]).reshape(_NW, _NCHUNK, _CHUNK)
    dst_idx = jnp.concatenate(
        [dst_idx, jnp.full((npad,), _NCELL, jnp.int32)]).reshape(_NW, _NCHUNK, _CHUNK)

    zeros = jnp.zeros((_ACC_PAD,), jnp.float32)
    ones = jnp.ones((_NCHUNK, _CHUNK), jnp.float32)
    acc, cnt = _make_sc_kernel()(s_flat, src_idx, dst_idx, zeros, ones)

    logits = _stage3(acc.reshape(_NC, _SIZE, _SIZE),
                     cnt.reshape(_NC, _SIZE, _SIZE), b)
    return logits[None, None, :, :]
